# baseline (speedup 1.0000x reference)
import functools

import jax
import jax.numpy as jnp
from jax import lax
from jax.experimental import pallas as pl
from jax.experimental.pallas import tpu as pltpu
from jax.experimental.pallas import tpu_sc as plsc

N = 10000
E = 320000
D = 128
DE = 16
L = 3

NC = 2
NS = 16
NW = NC * NS
EPT = E // NW
C = 80
NCH = EPT // C
NP = 10240
STRIPE = NP // NS

_mesh = plsc.VectorSubcoreMesh(core_axis_name="c", subcore_axis_name="s",
                               num_cores=NC, num_subcores=NS)
_sc_params = pltpu.CompilerParams(needs_layout_passes=False)


def _wid():
    return lax.axis_index("s") * NC + lax.axis_index("c")


def _fill_iota(idx_v, base):
    def grp(j, _):
        idx_v[pl.ds(j * 16, 16)] = (
            lax.broadcasted_iota(jnp.int32, (16,), 0) + j * 16 + base
        )
        return 0

    lax.fori_loop(0, C // 16, grp, 0)


def _deg_body(rowf_hbm, out_hbm, rowc_v, iz_v, ones_v, zb_v, acc_sh):
    c = lax.axis_index("c")
    s = lax.axis_index("s")
    wid = _wid()
    ebase = wid * EPT
    flat = c * NP + s * STRIPE

    def fill(i, _):
        ones_v[i, :] = jnp.full((16,), 1.0, jnp.float32)
        zb_v[i, :] = jnp.zeros((16,), jnp.float32)
        return 0

    lax.fori_loop(0, C, fill, 0)

    def zero(q, _):
        _fill_iota(iz_v, s * STRIPE + q * C)
        pltpu.sync_copy(zb_v, acc_sh.at[iz_v])
        return 0

    lax.fori_loop(0, STRIPE // C, zero, 0)
    plsc.subcore_barrier()

    def chunk(g, _):
        pltpu.sync_copy(rowf_hbm.at[pl.ds(ebase + g * C, C)], rowc_v)
        pltpu.sync_copy(ones_v, acc_sh.at[rowc_v], add=True)
        return 0

    lax.fori_loop(0, NCH, chunk, 0)
    plsc.subcore_barrier()

    def dump(q, _):
        pltpu.sync_copy(acc_sh.at[pl.ds(s * STRIPE + q * C, C)], zb_v)
        pltpu.sync_copy(zb_v, out_hbm.at[pl.ds(flat + q * C, C)])
        return 0

    lax.fori_loop(0, STRIPE // C, dump, 0)


_deg_call = pl.kernel(
    _deg_body,
    out_type=jax.ShapeDtypeStruct((NC * NP, 16), jnp.float32),
    mesh=_mesh,
    scratch_types=[
        pltpu.VMEM((C,), jnp.int32),
        pltpu.VMEM((C,), jnp.int32),
        pltpu.VMEM((C, 16), jnp.float32),
        pltpu.VMEM((C, 16), jnp.float32),
        pltpu.VMEM_SHARED((NP, 16), jnp.float32),
    ],
    compiler_params=_sc_params,
)


def _norm_body(rowf_hbm, colf_hbm, dis_hbm, out_hbm, rowf_v, colf_v, dis_v,
               norm_v):
    wid = _wid()
    ebase = wid * EPT
    pltpu.sync_copy(rowf_hbm.at[pl.ds(ebase, EPT)], rowf_v)
    pltpu.sync_copy(colf_hbm.at[pl.ds(ebase, EPT)], colf_v)
    pltpu.sync_copy(dis_hbm, dis_v)

    def norm_grp(j, _):
        rv = rowf_v[pl.ds(j * 16, 16)]
        cv = colf_v[pl.ds(j * 16, 16)]
        dr = plsc.load_gather(dis_v, [rv])
        dc = plsc.load_gather(dis_v, [cv])
        norm_v[pl.ds(j * 16, 16)] = dr * dc
        return 0

    lax.fori_loop(0, EPT // 16, norm_grp, 0)
    pltpu.sync_copy(norm_v, out_hbm.at[pl.ds(ebase, EPT)])


_norm_call = pl.kernel(
    _norm_body,
    out_type=jax.ShapeDtypeStruct((E,), jnp.float32),
    mesh=_mesh,
    scratch_types=[
        pltpu.VMEM((EPT,), jnp.int32),
        pltpu.VMEM((EPT,), jnp.int32),
        pltpu.VMEM((NP,), jnp.float32),
        pltpu.VMEM((EPT,), jnp.float32),
    ],
    compiler_params=_sc_params,
)


def _msg_body(hl_hbm, e_hbm, rowf_hbm, colf_hbm, norm_hbm, out_hbm,
              row0_v, col0_v, nrm0_v, row1_v, col1_v, nrm1_v, iz_v,
              hlr0_v, er0_v, hlr1_v, er1_v, acc_sh,
              semg0, seme0, semg1, seme1, semi0, semi1, sems0):
    c = lax.axis_index("c")
    s = lax.axis_index("s")
    wid = _wid()
    ebase = wid * EPT
    flat = c * NP + s * STRIPE

    def fill_zero(i, _):
        for k in range(8):
            er0_v[i, pl.ds(k * 16, 16)] = jnp.zeros((16,), jnp.float32)
        return 0

    lax.fori_loop(0, C, fill_zero, 0)

    def zero(q, _):
        _fill_iota(iz_v, s * STRIPE + q * C)
        pltpu.sync_copy(er0_v, acc_sh.at[iz_v])
        return 0

    lax.fori_loop(0, STRIPE // C, zero, 0)
    plsc.subcore_barrier()

    def idx_load(g, row_v, col_v, nrm_v, semi):
        base = ebase + g * C
        a = pltpu.async_copy(rowf_hbm.at[pl.ds(base, C)], row_v, semi)
        b = pltpu.async_copy(colf_hbm.at[pl.ds(base, C)], col_v, semi)
        d = pltpu.async_copy(norm_hbm.at[pl.ds(base, C)], nrm_v, semi)
        return a, b, d

    def data_load(g, row_v, hlr_v, er_v, semg, seme):
        ga = pltpu.async_copy(hl_hbm.at[row_v], hlr_v, semg)
        eb = pltpu.async_copy(e_hbm.at[pl.ds(ebase + g * C, C)], er_v, seme)
        return ga, eb

    def compute(nrm_v, hlr_v, er_v):
        def edge(i4, _):
            for u in range(4):
                i = i4 * 4 + u
                nsp = plsc.load_gather(nrm_v, [jnp.full((16,), i,
                                                        jnp.int32)])
                for k in range(8):
                    sl = pl.ds(k * 16, 16)
                    er_v[i, sl] = jnp.maximum(hlr_v[i, sl] + er_v[i, sl],
                                              0.0) * nsp
            return 0

        lax.fori_loop(0, C // 4, edge, 0)

    a, b, d = idx_load(0, row0_v, col0_v, nrm0_v, semi0)
    a.wait(); b.wait(); d.wait()
    ga, eb = data_load(0, row0_v, hlr0_v, er0_v, semg0, seme0)

    def pair(g, _):
        a0 = 2 * g
        b1 = 2 * g + 1
        ia, ib, ic = idx_load(b1, row1_v, col1_v, nrm1_v, semi1)
        ia.wait(); ib.wait(); ic.wait()
        ga1, eb1 = data_load(b1, row1_v, hlr1_v, er1_v, semg1, seme1)
        pltpu.make_async_copy(hl_hbm.at[row0_v], hlr0_v, semg0).wait()
        pltpu.make_async_copy(e_hbm.at[pl.ds(0, C)], er0_v, seme0).wait()
        compute(nrm0_v, hlr0_v, er0_v)
        pltpu.async_copy(er0_v, acc_sh.at[col0_v], sems0, add=True)
        ga1.wait()
        eb1.wait()
        compute(nrm1_v, hlr1_v, er1_v)
        pltpu.make_async_copy(er0_v, acc_sh.at[col0_v], sems0).wait()
        @pl.when(g < (NCH - 1) // 2)
        def _():
            ja, jb, jc = idx_load(a0 + 2, row0_v, col0_v, nrm0_v, semi0)
            ja.wait(); jb.wait(); jc.wait()
            data_load(a0 + 2, row0_v, hlr0_v, er0_v, semg0, seme0)

        pltpu.sync_copy(er1_v, acc_sh.at[col1_v], add=True)
        return 0

    lax.fori_loop(0, NCH // 2, pair, 0)
    pltpu.make_async_copy(hl_hbm.at[row0_v], hlr0_v, semg0).wait()
    pltpu.make_async_copy(e_hbm.at[pl.ds(0, C)], er0_v, seme0).wait()
    compute(nrm0_v, hlr0_v, er0_v)
    pltpu.sync_copy(er0_v, acc_sh.at[col0_v], add=True)

    plsc.subcore_barrier()

    def dump(q, _):
        pltpu.sync_copy(acc_sh.at[pl.ds(s * STRIPE + q * C, C)], er0_v)
        pltpu.sync_copy(er0_v, out_hbm.at[pl.ds(flat + q * C, C)])
        return 0

    lax.fori_loop(0, STRIPE // C, dump, 0)


_msg_call = pl.kernel(
    _msg_body,
    out_type=jax.ShapeDtypeStruct((NC * NP, D), jnp.float32),
    mesh=_mesh,
    scratch_types=[
        pltpu.VMEM((C,), jnp.int32),
        pltpu.VMEM((C,), jnp.int32),
        pltpu.VMEM((C,), jnp.float32),
        pltpu.VMEM((C,), jnp.int32),
        pltpu.VMEM((C,), jnp.int32),
        pltpu.VMEM((C,), jnp.float32),
        pltpu.VMEM((C,), jnp.int32),
        pltpu.VMEM((C, D), jnp.float32),
        pltpu.VMEM((C, D), jnp.float32),
        pltpu.VMEM((C, D), jnp.float32),
        pltpu.VMEM((C, D), jnp.float32),
        pltpu.VMEM_SHARED((NP, D), jnp.float32),
        pltpu.SemaphoreType.DMA,
        pltpu.SemaphoreType.DMA,
        pltpu.SemaphoreType.DMA,
        pltpu.SemaphoreType.DMA,
        pltpu.SemaphoreType.DMA,
        pltpu.SemaphoreType.DMA,
        pltpu.SemaphoreType.DMA,
    ],
    compiler_params=_sc_params,
)


def _prep_body(x_ref, nW_ref, nb_ref, W0_ref, b0_ref, deg16_ref,
               hl0_ref, dis_ref, dinv_ref):
    h = jnp.dot(x_ref[...], nW_ref[...],
                preferred_element_type=jnp.float32) + nb_ref[...]
    hl0_ref[...] = jnp.dot(h, W0_ref[...],
                           preferred_element_type=jnp.float32) + b0_ref[...]
    t = deg16_ref[...][:NP] + deg16_ref[...][NP:]
    deg = jnp.sum(t, axis=1, keepdims=True) * (1.0 / 16.0) + 1.0
    dis_ref[...] = lax.rsqrt(deg)
    dinv_ref[...] = (1.0 / deg)[:N]


_prep_call = pl.pallas_call(
    _prep_body,
    out_shape=[
        jax.ShapeDtypeStruct((N, D), jnp.float32),
        jax.ShapeDtypeStruct((NP, 1), jnp.float32),
        jax.ShapeDtypeStruct((N, 1), jnp.float32),
    ],
)


def _edge_body(ea_ref, eW_ref, eb_ref, o_ref):
    o_ref[...] = jnp.dot(ea_ref[...], eW_ref[...],
                         preferred_element_type=jnp.float32) + eb_ref[...]


BE = 6400
_edge_call = pl.pallas_call(
    _edge_body,
    grid=(E // BE,),
    in_specs=[
        pl.BlockSpec((BE, DE), lambda i: (i, 0)),
        pl.BlockSpec((DE, D), lambda i: (0, 0)),
        pl.BlockSpec((1, D), lambda i: (0, 0)),
    ],
    out_specs=pl.BlockSpec((BE, D), lambda i: (i, 0)),
    out_shape=jax.ShapeDtypeStruct((E, D), jnp.float32),
)


def _epi_body(agg2_ref, hl_ref, root_ref, dinv_ref, gamma_ref, beta_ref,
              *rest, act, nxt):
    a2 = agg2_ref[...]
    pre = (a2[:N] + a2[NP:NP + N]
           + jnp.maximum(hl_ref[...] + root_ref[...], 0.0) * dinv_ref[...])
    mean = jnp.mean(pre, axis=0, keepdims=True)
    var = jnp.mean(pre * pre, axis=0, keepdims=True) - mean * mean
    y = (pre - mean) * lax.rsqrt(var + 1e-5) * gamma_ref[...] + beta_ref[...]
    if act:
        y = jnp.maximum(y, 0.0)
    if nxt:
        Wn_ref, bn_ref, o_ref = rest
        o_ref[...] = jnp.dot(y, Wn_ref[...],
                             preferred_element_type=jnp.float32) + bn_ref[...]
    else:
        (o_ref,) = rest
        o_ref[...] = y


_epi_next_call = pl.pallas_call(
    functools.partial(_epi_body, act=True, nxt=True),
    out_shape=jax.ShapeDtypeStruct((N, D), jnp.float32),
)
_epi_last_call = pl.pallas_call(
    functools.partial(_epi_body, act=False, nxt=False),
    out_shape=jax.ShapeDtypeStruct((N, D), jnp.float32),
)


def kernel(x, edge_index, edge_attr, batch, node_W, node_b, W, b, eW, eb,
           root, gamma, beta):
    del batch
    row = edge_index[0]
    col = edge_index[1]

    deg16 = _deg_call(row)
    hl, dis, dinv = _prep_call(x, node_W, node_b[None, :], W[0], b[0][None, :],
                               deg16)
    norm = _norm_call(row, col, dis.reshape(NP))
    for l in range(L):
        e = _edge_call(edge_attr, eW[l], eb[l][None, :])
        agg2 = _msg_call(hl, e, row, col, norm)
        if l < L - 1:
            hl = _epi_next_call(agg2, hl, root[l][None, :], dinv,
                                gamma[l][None, :], beta[l][None, :],
                                W[l + 1], b[l + 1][None, :])
        else:
            out = _epi_last_call(agg2, hl, root[l][None, :], dinv,
                                 gamma[l][None, :], beta[l][None, :])
    return out


# speedup vs baseline: 1.0205x; 1.0205x over previous
import functools

import jax
import jax.numpy as jnp
from jax import lax
from jax.experimental import pallas as pl
from jax.experimental.pallas import tpu as pltpu
from jax.experimental.pallas import tpu_sc as plsc

N = 10000
E = 320000
D = 128
DE = 16
L = 3

NC = 2
NS = 16
NW = NC * NS
EPT = E // NW
C = 80
NCH = EPT // C
NP = 10240
STRIPE = NP // NS

_mesh = plsc.VectorSubcoreMesh(core_axis_name="c", subcore_axis_name="s",
                               num_cores=NC, num_subcores=NS)
_sc_params = pltpu.CompilerParams(needs_layout_passes=False)


def _wid():
    return lax.axis_index("s") * NC + lax.axis_index("c")


def _fill_iota(idx_v, base):
    def grp(j, _):
        idx_v[pl.ds(j * 16, 16)] = (
            lax.broadcasted_iota(jnp.int32, (16,), 0) + j * 16 + base
        )
        return 0

    lax.fori_loop(0, C // 16, grp, 0)


def _deg_body(rowf_hbm, out_hbm, rowc_v, iz_v, ones_v, zb_v, acc_sh):
    c = lax.axis_index("c")
    s = lax.axis_index("s")
    wid = _wid()
    ebase = wid * EPT
    flat = c * NP + s * STRIPE

    def fill(i, _):
        ones_v[i, :] = jnp.full((16,), 1.0, jnp.float32)
        zb_v[i, :] = jnp.zeros((16,), jnp.float32)
        return 0

    lax.fori_loop(0, C, fill, 0)

    def zero(q, _):
        _fill_iota(iz_v, s * STRIPE + q * C)
        pltpu.sync_copy(zb_v, acc_sh.at[iz_v])
        return 0

    lax.fori_loop(0, STRIPE // C, zero, 0)
    plsc.subcore_barrier()

    def chunk(g, _):
        pltpu.sync_copy(rowf_hbm.at[pl.ds(ebase + g * C, C)], rowc_v)
        pltpu.sync_copy(ones_v, acc_sh.at[rowc_v], add=True)
        return 0

    lax.fori_loop(0, NCH, chunk, 0)
    plsc.subcore_barrier()

    def dump(q, _):
        pltpu.sync_copy(acc_sh.at[pl.ds(s * STRIPE + q * C, C)], zb_v)
        pltpu.sync_copy(zb_v, out_hbm.at[pl.ds(flat + q * C, C)])
        return 0

    lax.fori_loop(0, STRIPE // C, dump, 0)


_deg_call = pl.kernel(
    _deg_body,
    out_type=jax.ShapeDtypeStruct((NC * NP, 16), jnp.float32),
    mesh=_mesh,
    scratch_types=[
        pltpu.VMEM((C,), jnp.int32),
        pltpu.VMEM((C,), jnp.int32),
        pltpu.VMEM((C, 16), jnp.float32),
        pltpu.VMEM((C, 16), jnp.float32),
        pltpu.VMEM_SHARED((NP, 16), jnp.float32),
    ],
    compiler_params=_sc_params,
)


def _norm_body(rowf_hbm, colf_hbm, dis_hbm, out_hbm, rowf_v, colf_v, dis_v,
               norm_v):
    wid = _wid()
    ebase = wid * EPT
    pltpu.sync_copy(rowf_hbm.at[pl.ds(ebase, EPT)], rowf_v)
    pltpu.sync_copy(colf_hbm.at[pl.ds(ebase, EPT)], colf_v)
    pltpu.sync_copy(dis_hbm, dis_v)

    def norm_grp(j, _):
        rv = rowf_v[pl.ds(j * 16, 16)]
        cv = colf_v[pl.ds(j * 16, 16)]
        dr = plsc.load_gather(dis_v, [rv])
        dc = plsc.load_gather(dis_v, [cv])
        norm_v[pl.ds(j * 16, 16)] = dr * dc
        return 0

    lax.fori_loop(0, EPT // 16, norm_grp, 0)
    pltpu.sync_copy(norm_v, out_hbm.at[pl.ds(ebase, EPT)])


_norm_call = pl.kernel(
    _norm_body,
    out_type=jax.ShapeDtypeStruct((E,), jnp.float32),
    mesh=_mesh,
    scratch_types=[
        pltpu.VMEM((EPT,), jnp.int32),
        pltpu.VMEM((EPT,), jnp.int32),
        pltpu.VMEM((NP,), jnp.float32),
        pltpu.VMEM((EPT,), jnp.float32),
    ],
    compiler_params=_sc_params,
)


def _msg_body(hl_hbm, e_hbm, rowf_hbm, colf_hbm, norm_hbm, out_hbm,
              row0_v, col0_v, nrm0_v, row1_v, col1_v, nrm1_v, iz_v,
              hlr0_v, er0_v, hlr1_v, er1_v, acc_sh,
              semg0, seme0, semg1, seme1, semi0, semi1):
    c = lax.axis_index("c")
    s = lax.axis_index("s")
    wid = _wid()
    ebase = wid * EPT
    flat = c * NP + s * STRIPE

    def fill_zero(i, _):
        for k in range(8):
            er0_v[i, pl.ds(k * 16, 16)] = jnp.zeros((16,), jnp.float32)
        return 0

    lax.fori_loop(0, C, fill_zero, 0)

    def zero(q, _):
        _fill_iota(iz_v, s * STRIPE + q * C)
        pltpu.sync_copy(er0_v, acc_sh.at[iz_v])
        return 0

    lax.fori_loop(0, STRIPE // C, zero, 0)
    plsc.subcore_barrier()

    def idx_load(g, row_v, col_v, nrm_v, semi):
        base = ebase + g * C
        a = pltpu.async_copy(rowf_hbm.at[pl.ds(base, C)], row_v, semi)
        b = pltpu.async_copy(colf_hbm.at[pl.ds(base, C)], col_v, semi)
        d = pltpu.async_copy(norm_hbm.at[pl.ds(base, C)], nrm_v, semi)
        return a, b, d

    def data_load(g, row_v, hlr_v, er_v, semg, seme):
        ga = pltpu.async_copy(hl_hbm.at[row_v], hlr_v, semg)
        eb = pltpu.async_copy(e_hbm.at[pl.ds(ebase + g * C, C)], er_v, seme)
        return ga, eb

    def compute(nrm_v, hlr_v, er_v):
        def edge(i, _):
            nsp = plsc.load_gather(nrm_v, [jnp.full((16,), i, jnp.int32)])
            for k in range(8):
                sl = pl.ds(k * 16, 16)
                er_v[i, sl] = jnp.maximum(hlr_v[i, sl] + er_v[i, sl],
                                          0.0) * nsp
            return 0

        lax.fori_loop(0, C, edge, 0)

    a, b, d = idx_load(0, row0_v, col0_v, nrm0_v, semi0)
    a.wait(); b.wait(); d.wait()
    ga, eb = data_load(0, row0_v, hlr0_v, er0_v, semg0, seme0)

    def pair(g, _):
        a0 = 2 * g
        b1 = 2 * g + 1
        ia, ib, ic = idx_load(b1, row1_v, col1_v, nrm1_v, semi1)
        ia.wait(); ib.wait(); ic.wait()
        ga1, eb1 = data_load(b1, row1_v, hlr1_v, er1_v, semg1, seme1)
        pltpu.make_async_copy(hl_hbm.at[row0_v], hlr0_v, semg0).wait()
        pltpu.make_async_copy(e_hbm.at[pl.ds(0, C)], er0_v, seme0).wait()
        compute(nrm0_v, hlr0_v, er0_v)
        pltpu.sync_copy(er0_v, acc_sh.at[col0_v], add=True)
        @pl.when(g < (NCH - 1) // 2)
        def _():
            ja, jb, jc = idx_load(a0 + 2, row0_v, col0_v, nrm0_v, semi0)
            ja.wait(); jb.wait(); jc.wait()
            data_load(a0 + 2, row0_v, hlr0_v, er0_v, semg0, seme0)

        ga1.wait()
        eb1.wait()
        compute(nrm1_v, hlr1_v, er1_v)
        pltpu.sync_copy(er1_v, acc_sh.at[col1_v], add=True)
        return 0

    lax.fori_loop(0, NCH // 2, pair, 0)
    pltpu.make_async_copy(hl_hbm.at[row0_v], hlr0_v, semg0).wait()
    pltpu.make_async_copy(e_hbm.at[pl.ds(0, C)], er0_v, seme0).wait()
    compute(nrm0_v, hlr0_v, er0_v)
    pltpu.sync_copy(er0_v, acc_sh.at[col0_v], add=True)

    plsc.subcore_barrier()

    def dump(q, _):
        pltpu.sync_copy(acc_sh.at[pl.ds(s * STRIPE + q * C, C)], er0_v)
        pltpu.sync_copy(er0_v, out_hbm.at[pl.ds(flat + q * C, C)])
        return 0

    lax.fori_loop(0, STRIPE // C, dump, 0)


_msg_call = pl.kernel(
    _msg_body,
    out_type=jax.ShapeDtypeStruct((NC * NP, D), jnp.float32),
    mesh=_mesh,
    scratch_types=[
        pltpu.VMEM((C,), jnp.int32),
        pltpu.VMEM((C,), jnp.int32),
        pltpu.VMEM((C,), jnp.float32),
        pltpu.VMEM((C,), jnp.int32),
        pltpu.VMEM((C,), jnp.int32),
        pltpu.VMEM((C,), jnp.float32),
        pltpu.VMEM((C,), jnp.int32),
        pltpu.VMEM((C, D), jnp.float32),
        pltpu.VMEM((C, D), jnp.float32),
        pltpu.VMEM((C, D), jnp.float32),
        pltpu.VMEM((C, D), jnp.float32),
        pltpu.VMEM_SHARED((NP, D), jnp.float32),
        pltpu.SemaphoreType.DMA,
        pltpu.SemaphoreType.DMA,
        pltpu.SemaphoreType.DMA,
        pltpu.SemaphoreType.DMA,
        pltpu.SemaphoreType.DMA,
        pltpu.SemaphoreType.DMA,
    ],
    compiler_params=_sc_params,
)


def _prep_body(x_ref, nW_ref, nb_ref, W0_ref, b0_ref, deg16_ref,
               hl0_ref, dis_ref, dinv_ref):
    h = jnp.dot(x_ref[...], nW_ref[...],
                preferred_element_type=jnp.float32) + nb_ref[...]
    hl0_ref[...] = jnp.dot(h, W0_ref[...],
                           preferred_element_type=jnp.float32) + b0_ref[...]
    t = deg16_ref[...][:NP] + deg16_ref[...][NP:]
    deg = jnp.sum(t, axis=1, keepdims=True) * (1.0 / 16.0) + 1.0
    dis_ref[...] = lax.rsqrt(deg)
    dinv_ref[...] = (1.0 / deg)[:N]


_prep_call = pl.pallas_call(
    _prep_body,
    out_shape=[
        jax.ShapeDtypeStruct((N, D), jnp.float32),
        jax.ShapeDtypeStruct((NP, 1), jnp.float32),
        jax.ShapeDtypeStruct((N, 1), jnp.float32),
    ],
)


def _edge_body(ea_ref, eW_ref, eb_ref, o_ref):
    o_ref[...] = jnp.dot(ea_ref[...], eW_ref[...],
                         preferred_element_type=jnp.float32) + eb_ref[...]


BE = 6400
_edge_call = pl.pallas_call(
    _edge_body,
    grid=(E // BE,),
    in_specs=[
        pl.BlockSpec((BE, DE), lambda i: (i, 0)),
        pl.BlockSpec((DE, D), lambda i: (0, 0)),
        pl.BlockSpec((1, D), lambda i: (0, 0)),
    ],
    out_specs=pl.BlockSpec((BE, D), lambda i: (i, 0)),
    out_shape=jax.ShapeDtypeStruct((E, D), jnp.float32),
)


def _epi_body(agg2_ref, hl_ref, root_ref, dinv_ref, gamma_ref, beta_ref,
              *rest, act, nxt):
    a2 = agg2_ref[...]
    pre = (a2[:N] + a2[NP:NP + N]
           + jnp.maximum(hl_ref[...] + root_ref[...], 0.0) * dinv_ref[...])
    mean = jnp.mean(pre, axis=0, keepdims=True)
    var = jnp.mean(pre * pre, axis=0, keepdims=True) - mean * mean
    y = (pre - mean) * lax.rsqrt(var + 1e-5) * gamma_ref[...] + beta_ref[...]
    if act:
        y = jnp.maximum(y, 0.0)
    if nxt:
        Wn_ref, bn_ref, o_ref = rest
        o_ref[...] = jnp.dot(y, Wn_ref[...],
                             preferred_element_type=jnp.float32) + bn_ref[...]
    else:
        (o_ref,) = rest
        o_ref[...] = y


_epi_next_call = pl.pallas_call(
    functools.partial(_epi_body, act=True, nxt=True),
    out_shape=jax.ShapeDtypeStruct((N, D), jnp.float32),
)
_epi_last_call = pl.pallas_call(
    functools.partial(_epi_body, act=False, nxt=False),
    out_shape=jax.ShapeDtypeStruct((N, D), jnp.float32),
)


def kernel(x, edge_index, edge_attr, batch, node_W, node_b, W, b, eW, eb,
           root, gamma, beta):
    del batch
    row = edge_index[0]
    col = edge_index[1]

    deg16 = _deg_call(row)
    hl, dis, dinv = _prep_call(x, node_W, node_b[None, :], W[0], b[0][None, :],
                               deg16)
    norm = _norm_call(row, col, dis.reshape(NP))
    for l in range(L):
        e = _edge_call(edge_attr, eW[l], eb[l][None, :])
        agg2 = _msg_call(hl, e, row, col, norm)
        if l < L - 1:
            hl = _epi_next_call(agg2, hl, root[l][None, :], dinv,
                                gamma[l][None, :], beta[l][None, :],
                                W[l + 1], b[l + 1][None, :])
        else:
            out = _epi_last_call(agg2, hl, root[l][None, :], dinv,
                                 gamma[l][None, :], beta[l][None, :])
    return out
